# Initial kernel scaffold; baseline (speedup 1.0000x reference)
import jax
import jax.numpy as jnp
from jax.experimental import pallas as pl


def kernel(x, rbf, idx_i, num_nodes, W_rbf, W0, b0, W1, b1, W2, b2, W_out):
    raise NotImplementedError("write your pallas kernel here")


# speedup vs baseline: 2.0304x; 2.0304x over previous
import functools

import jax
import jax.numpy as jnp
from jax import lax
from jax.experimental import pallas as pl
from jax.experimental.pallas import tpu as pltpu
from jax.experimental.pallas import tpu_sc as plsc

_N_EDGES = 320000
_N_NODES = 10000
_EDGE_DIM = 128
_NRAD_PAD = 8

_NC = 2
_NS = 16
_EDGES_PER_TILE = _N_EDGES // (_NC * _NS)
_CHUNK = 80
_STEPS = _EDGES_PER_TILE // _CHUNK
_ZRT = 624
_ZRT_LAST = _N_NODES - (_NS - 1) * _ZRT

_EB = 4000
_NB = 2000


def _gate_body(rbf_ref, x_ref, wT_ref, h_ref):
    g = jnp.dot(rbf_ref[...], wT_ref[...], preferred_element_type=jnp.float32)
    h_ref[...] = g * x_ref[...]


def _edge_gate(rbf8, x, w8T):
    return pl.pallas_call(
        _gate_body,
        grid=(_N_EDGES // _EB,),
        in_specs=[
            pl.BlockSpec((_EB, _NRAD_PAD), lambda i: (i, 0)),
            pl.BlockSpec((_EB, _EDGE_DIM), lambda i: (i, 0)),
            pl.BlockSpec((_NRAD_PAD, _EDGE_DIM), lambda i: (0, 0)),
        ],
        out_specs=pl.BlockSpec((_EB, _EDGE_DIM), lambda i: (i, 0)),
        out_shape=jax.ShapeDtypeStruct((_N_EDGES, _EDGE_DIM), jnp.float32),
    )(rbf8, x, w8T)


def _sc_scatter(h, idx, zrows):
    mesh = plsc.VectorSubcoreMesh(core_axis_name="c", subcore_axis_name="s")

    @functools.partial(
        pl.kernel,
        mesh=mesh,
        out_type=jax.ShapeDtypeStruct((_NC, _N_NODES, _EDGE_DIM), jnp.float32),
        scratch_types=[
            pltpu.VMEM((_CHUNK, _EDGE_DIM), jnp.float32),
            pltpu.VMEM((_CHUNK,), jnp.int32),
            pltpu.VMEM_SHARED((_N_NODES, _EDGE_DIM), jnp.float32),
        ],
    )
    def scat(h_hbm, idx_hbm, z_hbm, out_hbm, h_v, idx_v, s_sh):
        c = lax.axis_index("c")
        s = lax.axis_index("s")
        base = (c * _NS + s) * _EDGES_PER_TILE
        r0 = s * _ZRT

        @pl.when(s < _NS - 1)
        def _():
            pltpu.sync_copy(z_hbm.at[pl.ds(0, _ZRT)], s_sh.at[pl.ds(r0, _ZRT)])

        @pl.when(s == _NS - 1)
        def _():
            pltpu.sync_copy(
                z_hbm, s_sh.at[pl.ds((_NS - 1) * _ZRT, _ZRT_LAST)])

        plsc.subcore_barrier()

        def body(i, carry):
            off = base + i * _CHUNK
            pltpu.sync_copy(h_hbm.at[pl.ds(off, _CHUNK)], h_v)
            pltpu.sync_copy(idx_hbm.at[pl.ds(off, _CHUNK)], idx_v)
            pltpu.sync_copy(h_v, s_sh.at[idx_v], add=True)
            return carry

        lax.fori_loop(0, _STEPS, body, 0)
        plsc.subcore_barrier()

        @pl.when(s < _NS - 1)
        def _():
            pltpu.sync_copy(s_sh.at[pl.ds(r0, _ZRT)],
                            out_hbm.at[c, pl.ds(r0, _ZRT)])

        @pl.when(s == _NS - 1)
        def _():
            pltpu.sync_copy(
                s_sh.at[pl.ds((_NS - 1) * _ZRT, _ZRT_LAST)],
                out_hbm.at[c, pl.ds((_NS - 1) * _ZRT, _ZRT_LAST)])

    return scat(h, idx, zrows)


def _sigmoid(v):
    return 1.0 / (1.0 + jnp.exp(-v))


def _mlp_body(s0_ref, s1_ref, w0_ref, b0_ref, w1_ref, b1_ref, w2_ref, b2_ref,
              wo_ref, o_ref):
    z = s0_ref[...] + s1_ref[...]
    z = jnp.dot(z, w0_ref[...], preferred_element_type=jnp.float32) + b0_ref[...]
    z = z * _sigmoid(z)
    z = jnp.dot(z, w1_ref[...], preferred_element_type=jnp.float32) + b1_ref[...]
    z = z * _sigmoid(z)
    z = jnp.dot(z, w2_ref[...], preferred_element_type=jnp.float32) + b2_ref[...]
    z = z * _sigmoid(z)
    o_ref[...] = jnp.dot(z, wo_ref[...], preferred_element_type=jnp.float32)


def _node_mlp(s0, s1, w0T, b0, w1T, b1, w2T, b2, woT):
    full = lambda r, c: pl.BlockSpec((r, c), lambda i: (0, 0))
    return pl.pallas_call(
        _mlp_body,
        grid=(_N_NODES // _NB,),
        in_specs=[
            pl.BlockSpec((_NB, _EDGE_DIM), lambda i: (i, 0)),
            pl.BlockSpec((_NB, _EDGE_DIM), lambda i: (i, 0)),
            full(_EDGE_DIM, _EDGE_DIM), full(1, _EDGE_DIM),
            full(_EDGE_DIM, _EDGE_DIM), full(1, _EDGE_DIM),
            full(_EDGE_DIM, _EDGE_DIM), full(1, _EDGE_DIM),
            full(_EDGE_DIM, 1),
        ],
        out_specs=pl.BlockSpec((_NB, 1), lambda i: (i, 0)),
        out_shape=jax.ShapeDtypeStruct((_N_NODES, 1), jnp.float32),
    )(s0, s1, w0T, b0, w1T, b1, w2T, b2, woT)


def kernel(x, rbf, idx_i, num_nodes, W_rbf, W0, b0, W1, b1, W2, b2, W_out):
    rbf8 = jnp.pad(rbf, ((0, 0), (0, _NRAD_PAD - rbf.shape[1])))
    w8T = jnp.pad(W_rbf.T, ((0, _NRAD_PAD - rbf.shape[1]), (0, 0)))
    idx32 = jnp.minimum(idx_i, num_nodes - 1).astype(jnp.int32)
    zrows = jnp.zeros((_ZRT_LAST, _EDGE_DIM), jnp.float32)

    h = _edge_gate(rbf8, x, w8T)
    sp = _sc_scatter(h, idx32, zrows)
    out = _node_mlp(
        sp[0], sp[1],
        W0.T, b0.reshape(1, -1),
        W1.T, b1.reshape(1, -1),
        W2.T, b2.reshape(1, -1),
        W_out.T,
    )
    return out


# speedup vs baseline: 2.7648x; 1.3617x over previous
import functools

import jax
import jax.numpy as jnp
from jax import lax
from jax.experimental import pallas as pl
from jax.experimental.pallas import tpu as pltpu
from jax.experimental.pallas import tpu_sc as plsc

_N_EDGES = 320000
_N_NODES = 10000
_EDGE_DIM = 128
_NRAD_PAD = 8

_NC = 2
_NS = 16
_EDGES_PER_TILE = _N_EDGES // (_NC * _NS)
_CHUNK = 80
_STEPS = _EDGES_PER_TILE // _CHUNK
_ZRT = 624
_ZRT_LAST = _N_NODES - (_NS - 1) * _ZRT

_EB = 8000
_NB = 2000


def _gate_body(rbf_ref, x_ref, wT_ref, h_ref):
    g = jnp.dot(rbf_ref[...], wT_ref[...], preferred_element_type=jnp.float32)
    h_ref[...] = g * x_ref[...]


def _edge_gate(rbf8, x, w8T):
    return pl.pallas_call(
        _gate_body,
        grid=(_N_EDGES // _EB,),
        in_specs=[
            pl.BlockSpec((_EB, _NRAD_PAD), lambda i: (i, 0)),
            pl.BlockSpec((_EB, _EDGE_DIM), lambda i: (i, 0)),
            pl.BlockSpec((_NRAD_PAD, _EDGE_DIM), lambda i: (0, 0)),
        ],
        out_specs=pl.BlockSpec((_EB, _EDGE_DIM), lambda i: (i, 0)),
        out_shape=jax.ShapeDtypeStruct((_N_EDGES, _EDGE_DIM), jnp.float32),
    )(rbf8, x, w8T)


def _sc_scatter(h, idx, zrows):
    mesh = plsc.VectorSubcoreMesh(core_axis_name="c", subcore_axis_name="s")

    @functools.partial(
        pl.kernel,
        mesh=mesh,
        out_type=(
            jax.ShapeDtypeStruct((_N_NODES, _EDGE_DIM), jnp.float32),
            jax.ShapeDtypeStruct((_N_NODES, _EDGE_DIM), jnp.float32),
        ),
        scratch_types=[
            pltpu.VMEM((_CHUNK, _EDGE_DIM), jnp.float32),
            pltpu.VMEM((_CHUNK, _EDGE_DIM), jnp.float32),
            pltpu.VMEM((_CHUNK,), jnp.int32),
            pltpu.VMEM((_CHUNK,), jnp.int32),
            pltpu.VMEM_SHARED((_N_NODES, _EDGE_DIM), jnp.float32),
            pltpu.SemaphoreType.DMA,
            pltpu.SemaphoreType.DMA,
        ],
    )
    def scat(h_hbm, idx_hbm, z_hbm, out0_hbm, out1_hbm, h_a, h_b, i_a, i_b,
             s_sh, sem_a, sem_b):
        c = lax.axis_index("c")
        s = lax.axis_index("s")
        base = (c * _NS + s) * _EDGES_PER_TILE

        def _fetch(k, hv, iv, sem):
            off = base + k * _CHUNK
            pltpu.async_copy(h_hbm.at[pl.ds(off, _CHUNK)], hv, sem)
            pltpu.async_copy(idx_hbm.at[pl.ds(off, _CHUNK)], iv, sem)

        def _drain(hv, iv, sem):
            pltpu.make_async_copy(h_hbm.at[pl.ds(0, _CHUNK)], hv, sem).wait()
            pltpu.make_async_copy(idx_hbm.at[pl.ds(0, _CHUNK)], iv, sem).wait()
        r0 = s * _ZRT

        @pl.when(s < _NS - 1)
        def _():
            pltpu.sync_copy(z_hbm.at[pl.ds(0, _ZRT)], s_sh.at[pl.ds(r0, _ZRT)])

        @pl.when(s == _NS - 1)
        def _():
            pltpu.sync_copy(
                z_hbm, s_sh.at[pl.ds((_NS - 1) * _ZRT, _ZRT_LAST)])

        plsc.subcore_barrier()

        _fetch(0, h_a, i_a, sem_a)

        def body(k, carry):
            c0 = 2 * k
            _fetch(c0 + 1, h_b, i_b, sem_b)
            _drain(h_a, i_a, sem_a)
            pltpu.sync_copy(h_a, s_sh.at[i_a], add=True)
            _fetch(c0 + 2, h_a, i_a, sem_a)
            _drain(h_b, i_b, sem_b)
            pltpu.sync_copy(h_b, s_sh.at[i_b], add=True)
            return carry

        lax.fori_loop(0, (_STEPS - 1) // 2, body, 0)
        _drain(h_a, i_a, sem_a)
        pltpu.sync_copy(h_a, s_sh.at[i_a], add=True)
        plsc.subcore_barrier()

        @pl.when(s < _NS - 1)
        def _():
            @pl.when(c == 0)
            def _():
                pltpu.sync_copy(s_sh.at[pl.ds(r0, _ZRT)],
                                out0_hbm.at[pl.ds(r0, _ZRT)])

            @pl.when(c == 1)
            def _():
                pltpu.sync_copy(s_sh.at[pl.ds(r0, _ZRT)],
                                out1_hbm.at[pl.ds(r0, _ZRT)])

        @pl.when(s == _NS - 1)
        def _():
            @pl.when(c == 0)
            def _():
                pltpu.sync_copy(
                    s_sh.at[pl.ds((_NS - 1) * _ZRT, _ZRT_LAST)],
                    out0_hbm.at[pl.ds((_NS - 1) * _ZRT, _ZRT_LAST)])

            @pl.when(c == 1)
            def _():
                pltpu.sync_copy(
                    s_sh.at[pl.ds((_NS - 1) * _ZRT, _ZRT_LAST)],
                    out1_hbm.at[pl.ds((_NS - 1) * _ZRT, _ZRT_LAST)])

    return scat(h, idx, zrows)


def _sigmoid(v):
    return 1.0 / (1.0 + jnp.exp(-v))


def _mlp_body(s0_ref, s1_ref, w0_ref, b0_ref, w1_ref, b1_ref, w2_ref, b2_ref,
              wo_ref, o_ref):
    z = s0_ref[...] + s1_ref[...]
    z = jnp.dot(z, w0_ref[...], preferred_element_type=jnp.float32) + b0_ref[...]
    z = z * _sigmoid(z)
    z = jnp.dot(z, w1_ref[...], preferred_element_type=jnp.float32) + b1_ref[...]
    z = z * _sigmoid(z)
    z = jnp.dot(z, w2_ref[...], preferred_element_type=jnp.float32) + b2_ref[...]
    z = z * _sigmoid(z)
    o_ref[...] = jnp.dot(z, wo_ref[...], preferred_element_type=jnp.float32)


def _node_mlp(s0, s1, w0T, b0, w1T, b1, w2T, b2, woT):
    full = lambda r, c: pl.BlockSpec((r, c), lambda i: (0, 0))
    return pl.pallas_call(
        _mlp_body,
        grid=(_N_NODES // _NB,),
        in_specs=[
            pl.BlockSpec((_NB, _EDGE_DIM), lambda i: (i, 0)),
            pl.BlockSpec((_NB, _EDGE_DIM), lambda i: (i, 0)),
            full(_EDGE_DIM, _EDGE_DIM), full(1, _EDGE_DIM),
            full(_EDGE_DIM, _EDGE_DIM), full(1, _EDGE_DIM),
            full(_EDGE_DIM, _EDGE_DIM), full(1, _EDGE_DIM),
            full(_EDGE_DIM, 1),
        ],
        out_specs=pl.BlockSpec((_NB, 1), lambda i: (i, 0)),
        out_shape=jax.ShapeDtypeStruct((_N_NODES, 1), jnp.float32),
    )(s0, s1, w0T, b0, w1T, b1, w2T, b2, woT)


def kernel(x, rbf, idx_i, num_nodes, W_rbf, W0, b0, W1, b1, W2, b2, W_out):
    rbf8 = jnp.pad(rbf, ((0, 0), (0, _NRAD_PAD - rbf.shape[1])))
    w8T = jnp.pad(W_rbf.T, ((0, _NRAD_PAD - rbf.shape[1]), (0, 0)))
    idx32 = jnp.minimum(idx_i, num_nodes - 1).astype(jnp.int32)
    zrows = jnp.zeros((_ZRT_LAST, _EDGE_DIM), jnp.float32)

    h = _edge_gate(rbf8, x, w8T)
    s0, s1 = _sc_scatter(h, idx32, zrows)
    out = _node_mlp(
        s0, s1,
        W0.T, b0.reshape(1, -1),
        W1.T, b1.reshape(1, -1),
        W2.T, b2.reshape(1, -1),
        W_out.T,
    )
    return out
